# Initial kernel scaffold; baseline (speedup 1.0000x reference)
import jax
import jax.numpy as jnp
from jax.experimental import pallas as pl


def kernel(x, edge_index, W1, b1, W2, b2):
    raise NotImplementedError("write your pallas kernel here")


# speedup vs baseline: 8.0886x; 8.0886x over previous
import dataclasses
import functools

import jax
import jax.numpy as jnp
from jax import lax
from jax.experimental import pallas as pl
from jax.experimental.pallas import tpu as pltpu
from jax.experimental.pallas import tpu_sc as plsc

N = 10000
D = 128
E = 320000

NC = 2
NS = 16
NW = NC * NS

NPAD = 10240
EPAD = 327680
CH = 128
EPW = EPAD // NW
BIG = 2048
NBIG = EPW // BIG
ROWS_PT = NPAD // NS


def _mesh():
    return plsc.VectorSubcoreMesh(core_axis_name="c", subcore_axis_name="s")


def _sc_params():
    cp = pltpu.CompilerParams()
    if "needs_layout_passes" in pltpu.CompilerParams.__dataclass_fields__:
        cp = dataclasses.replace(cp, needs_layout_passes=False)
    return cp


@functools.cache
def _make_sc_hist():
    @functools.partial(
        pl.kernel,
        out_type=jax.ShapeDtypeStruct((NW, NPAD), jnp.float32),
        mesh=_mesh(),
        compiler_params=_sc_params(),
        scratch_types=[
            pltpu.VMEM((NPAD,), jnp.float32),
            pltpu.VMEM((BIG,), jnp.int32),
        ],
    )
    def sc_hist(dst_hbm, out_hbm, hist, dbuf):
        cid = lax.axis_index("c")
        sid = lax.axis_index("s")
        wid = cid * NS + sid

        @pl.loop(0, NPAD, step=16)
        def _(i):
            hist[pl.ds(i, 16)] = jnp.zeros((16,), jnp.float32)

        ebase = wid * EPW
        ones = jnp.ones((16,), jnp.float32)

        @pl.loop(0, NBIG)
        def _(j):
            pltpu.sync_copy(dst_hbm.at[pl.ds(ebase + j * BIG, BIG)], dbuf)

            @pl.loop(0, BIG, step=16)
            def _(i):
                plsc.addupdate_scatter(hist, [dbuf[pl.ds(i, 16)]], ones)

        pltpu.sync_copy(hist, out_hbm.at[wid])

    return sc_hist


@functools.cache
def _make_sc_edge():
    @functools.partial(
        pl.kernel,
        out_type=jax.ShapeDtypeStruct((NC, NPAD, D), jnp.float32),
        mesh=_mesh(),
        scratch_types=[
            pltpu.VMEM((BIG,), jnp.int32),
            pltpu.VMEM((BIG // CH, CH), jnp.int32),
            pltpu.VMEM((CH, D), jnp.float32),
            pltpu.VMEM_SHARED((NPAD, D), jnp.float32),
        ],
    )
    def sc_edge(hs_hbm, src_hbm, dst2_hbm, out_hbm, sbuf, dbuf, rows, acc):
        cid = lax.axis_index("c")
        sid = lax.axis_index("s")
        wid = cid * NS + sid

        @pl.loop(0, CH)
        def _(r):
            @pl.loop(0, D, step=16)
            def _(c):
                rows[r, pl.ds(c, 16)] = jnp.zeros((16,), jnp.float32)

        abase = sid * ROWS_PT

        @pl.loop(0, ROWS_PT // CH)
        def _(z):
            pltpu.sync_copy(rows, acc.at[pl.ds(abase + z * CH, CH)])

        plsc.subcore_barrier()

        ebase = wid * EPW
        rbase = wid * (EPW // CH)

        @pl.loop(0, NBIG)
        def _(j):
            pltpu.sync_copy(src_hbm.at[pl.ds(ebase + j * BIG, BIG)], sbuf)
            pltpu.sync_copy(
                dst2_hbm.at[pl.ds(rbase + j * (BIG // CH), BIG // CH)], dbuf)

            @pl.loop(0, BIG // CH)
            def _(k):
                pltpu.sync_copy(hs_hbm.at[sbuf.at[pl.ds(k * CH, CH)]], rows)
                pltpu.sync_copy(rows, acc.at[dbuf.at[k]], add=True)

        plsc.subcore_barrier()
        pltpu.sync_copy(acc.at[pl.ds(abase, ROWS_PT)],
                        out_hbm.at[cid, pl.ds(abase, ROWS_PT)])

    return sc_edge


def _dinv_from_parts(parts_t):
    deg = jnp.sum(parts_t, axis=1, keepdims=True) + 1.0
    return 1.0 / jnp.sqrt(deg)


def _tc_pre_body(pt_ref, xp_ref, w_ref, hs_ref, hd_ref):
    dinv = _dinv_from_parts(pt_ref[...])
    h = jnp.dot(xp_ref[...], w_ref[...], preferred_element_type=jnp.float32)
    hs = h * dinv
    hs_ref[...] = hs
    hd_ref[...] = hs * dinv


def _tc_mid_body(a_ref, b_ref, pt_ref, hd_ref, b1_ref, w_ref, hs_ref, hd2_ref):
    dinv = _dinv_from_parts(pt_ref[...])
    t = (a_ref[...] + b_ref[...]) * dinv + hd_ref[...] + b1_ref[...]
    n = jnp.sqrt(jnp.sum(t * t, axis=1, keepdims=True))
    x1 = t / jnp.maximum(n, 1e-12)
    h2 = jnp.dot(x1, w_ref[...], preferred_element_type=jnp.float32)
    hs2 = h2 * dinv
    hs_ref[...] = hs2
    hd2_ref[...] = hs2 * dinv


def _tc_final_body(a_ref, b_ref, pt_ref, hd_ref, b2_ref, xp_ref, out_ref):
    dinv = _dinv_from_parts(pt_ref[...])
    out_ref[...] = ((a_ref[...] + b_ref[...]) * dinv + hd_ref[...]
                    + b2_ref[...] + xp_ref[...])


_nd = jax.ShapeDtypeStruct((NPAD, D), jnp.float32)

_tc_pre = pl.pallas_call(_tc_pre_body, out_shape=(_nd, _nd))
_tc_mid = pl.pallas_call(_tc_mid_body, out_shape=(_nd, _nd))
_tc_final = pl.pallas_call(_tc_final_body, out_shape=_nd)


def kernel(x, edge_index, W1, b1, W2, b2):
    src = jnp.asarray(edge_index[0], jnp.int32)
    dst = jnp.asarray(edge_index[1], jnp.int32)
    pad = EPAD - E
    srcp = jnp.concatenate([src, jnp.zeros((pad,), jnp.int32)])
    dstp = jnp.concatenate([dst, jnp.full((pad,), NPAD - 1, jnp.int32)])
    dst2 = dstp.reshape(EPAD // CH, CH)
    xp = jnp.pad(x, ((0, NPAD - N), (0, 0)))
    b1r = b1.reshape(1, D)
    b2r = b2.reshape(1, D)

    sc_hist = _make_sc_hist()
    sc_edge = _make_sc_edge()

    parts = sc_hist(dstp)
    parts_t = parts.T

    hs1, hd1 = _tc_pre(parts_t, xp, W1)
    acc1 = sc_edge(hs1, srcp, dst2)
    hs2, hd2 = _tc_mid(acc1[0], acc1[1], parts_t, hd1, b1r, W2)
    acc2 = sc_edge(hs2, srcp, dst2)
    outp = _tc_final(acc2[0], acc2[1], parts_t, hd2, b2r, xp)
    return outp[:N]


# speedup vs baseline: 8.3515x; 1.0325x over previous
import dataclasses
import functools

import jax
import jax.numpy as jnp
from jax import lax
from jax.experimental import pallas as pl
from jax.experimental.pallas import tpu as pltpu
from jax.experimental.pallas import tpu_sc as plsc

N = 10000
D = 128
E = 320000

NC = 2
NS = 16
NW = NC * NS

NPAD = 10240
EPAD = 327680
CH = 128
EPW = EPAD // NW
BIG = 2048
NBIG = EPW // BIG
ROWS_PT = NPAD // NS


def _mesh():
    return plsc.VectorSubcoreMesh(core_axis_name="c", subcore_axis_name="s")


def _sc_params():
    cp = pltpu.CompilerParams()
    if "needs_layout_passes" in pltpu.CompilerParams.__dataclass_fields__:
        cp = dataclasses.replace(cp, needs_layout_passes=False)
    return cp


@functools.cache
def _make_sc_hist():
    @functools.partial(
        pl.kernel,
        out_type=jax.ShapeDtypeStruct((NW, NPAD), jnp.float32),
        mesh=_mesh(),
        compiler_params=_sc_params(),
        scratch_types=[
            pltpu.VMEM((NPAD,), jnp.float32),
            pltpu.VMEM((BIG,), jnp.int32),
        ],
    )
    def sc_hist(dst_hbm, out_hbm, hist, dbuf):
        cid = lax.axis_index("c")
        sid = lax.axis_index("s")
        wid = cid * NS + sid

        @pl.loop(0, NPAD, step=16)
        def _(i):
            hist[pl.ds(i, 16)] = jnp.zeros((16,), jnp.float32)

        ebase = wid * EPW
        ones = jnp.ones((16,), jnp.float32)

        @pl.loop(0, NBIG)
        def _(j):
            pltpu.sync_copy(dst_hbm.at[pl.ds(ebase + j * BIG, BIG)], dbuf)

            @pl.loop(0, BIG, step=16)
            def _(i):
                plsc.addupdate_scatter(hist, [dbuf[pl.ds(i, 16)]], ones)

        pltpu.sync_copy(hist, out_hbm.at[wid])

    return sc_hist


NBUF = 2
CPW = EPW // CH
BCH = BIG // CH
NGRP = BCH // NBUF


@functools.cache
def _make_sc_edge():
    @functools.partial(
        pl.kernel,
        out_type=jax.ShapeDtypeStruct((NC, NPAD, D), jnp.float32),
        mesh=_mesh(),
        scratch_types=(
            [pltpu.VMEM((BIG,), jnp.int32)]
            + [pltpu.VMEM((BCH, CH), jnp.int32)]
            + [pltpu.VMEM((CH, D), jnp.float32)] * NBUF
            + [pltpu.VMEM_SHARED((NPAD, D), jnp.float32)]
            + [pltpu.SemaphoreType.DMA] * (2 * NBUF)
        ),
    )
    def sc_edge(hs_hbm, src_hbm, dst2_hbm, out_hbm, sbuf, dbuf, *rest):
        rows = rest[:NBUF]
        acc = rest[NBUF]
        gsem = rest[NBUF + 1:NBUF + 1 + NBUF]
        ssem = rest[NBUF + 1 + NBUF:]
        cid = lax.axis_index("c")
        sid = lax.axis_index("s")
        wid = cid * NS + sid

        @pl.loop(0, CH)
        def _(r):
            @pl.loop(0, D, step=16)
            def _(c):
                rows[0][r, pl.ds(c, 16)] = jnp.zeros((16,), jnp.float32)

        abase = sid * ROWS_PT

        @pl.loop(0, ROWS_PT // CH)
        def _(z):
            pltpu.sync_copy(rows[0], acc.at[pl.ds(abase + z * CH, CH)])

        plsc.subcore_barrier()

        ebase = wid * EPW
        rbase = wid * CPW

        @pl.loop(0, NBIG)
        def _(j):
            pltpu.sync_copy(src_hbm.at[pl.ds(ebase + j * BIG, BIG)], sbuf)
            pltpu.sync_copy(dst2_hbm.at[pl.ds(rbase + j * BCH, BCH)], dbuf)

            @pl.loop(0, NGRP)
            def _(g):
                c0 = g * NBUF
                gd = [
                    pltpu.async_copy(
                        hs_hbm.at[sbuf.at[pl.ds((c0 + b) * CH, CH)]],
                        rows[b], gsem[b])
                    for b in range(NBUF)
                ]
                sd = []
                for b in range(NBUF):
                    gd[b].wait()
                    sd.append(pltpu.async_copy(
                        rows[b], acc.at[dbuf.at[c0 + b]], ssem[b], add=True))
                for b in range(NBUF):
                    sd[b].wait()

        plsc.subcore_barrier()
        pltpu.sync_copy(acc.at[pl.ds(abase, ROWS_PT)],
                        out_hbm.at[cid, pl.ds(abase, ROWS_PT)])

    return sc_edge


def _dinv_from_parts(parts_t):
    deg = jnp.sum(parts_t, axis=1, keepdims=True) + 1.0
    return 1.0 / jnp.sqrt(deg)


def _tc_pre_body(pt_ref, xp_ref, w_ref, hs_ref, hd_ref):
    dinv = _dinv_from_parts(pt_ref[...])
    h = jnp.dot(xp_ref[...], w_ref[...], preferred_element_type=jnp.float32)
    hs = h * dinv
    hs_ref[...] = hs
    hd_ref[...] = hs * dinv


def _tc_mid_body(a_ref, b_ref, pt_ref, hd_ref, b1_ref, w_ref, hs_ref, hd2_ref):
    dinv = _dinv_from_parts(pt_ref[...])
    t = (a_ref[...] + b_ref[...]) * dinv + hd_ref[...] + b1_ref[...]
    n = jnp.sqrt(jnp.sum(t * t, axis=1, keepdims=True))
    x1 = t / jnp.maximum(n, 1e-12)
    h2 = jnp.dot(x1, w_ref[...], preferred_element_type=jnp.float32)
    hs2 = h2 * dinv
    hs_ref[...] = hs2
    hd2_ref[...] = hs2 * dinv


def _tc_final_body(a_ref, b_ref, pt_ref, hd_ref, b2_ref, xp_ref, out_ref):
    dinv = _dinv_from_parts(pt_ref[...])
    out_ref[...] = ((a_ref[...] + b_ref[...]) * dinv + hd_ref[...]
                    + b2_ref[...] + xp_ref[...])


_nd = jax.ShapeDtypeStruct((NPAD, D), jnp.float32)

_tc_pre = pl.pallas_call(_tc_pre_body, out_shape=(_nd, _nd))
_tc_mid = pl.pallas_call(_tc_mid_body, out_shape=(_nd, _nd))
_tc_final = pl.pallas_call(_tc_final_body, out_shape=_nd)


def kernel(x, edge_index, W1, b1, W2, b2):
    src = jnp.asarray(edge_index[0], jnp.int32)
    dst = jnp.asarray(edge_index[1], jnp.int32)
    pad = EPAD - E
    srcp = jnp.concatenate([src, jnp.zeros((pad,), jnp.int32)])
    dstp = jnp.concatenate([dst, jnp.full((pad,), NPAD - 1, jnp.int32)])
    dst2 = dstp.reshape(EPAD // CH, CH)
    xp = jnp.pad(x, ((0, NPAD - N), (0, 0)))
    b1r = b1.reshape(1, D)
    b2r = b2.reshape(1, D)

    sc_hist = _make_sc_hist()
    sc_edge = _make_sc_edge()

    parts = sc_hist(dstp)
    parts_t = parts.T

    hs1, hd1 = _tc_pre(parts_t, xp, W1)
    acc1 = sc_edge(hs1, srcp, dst2)
    hs2, hd2 = _tc_mid(acc1[0], acc1[1], parts_t, hd1, b1r, W2)
    acc2 = sc_edge(hs2, srcp, dst2)
    outp = _tc_final(acc2[0], acc2[1], parts_t, hd2, b2r, xp)
    return outp[:N]
